# baseline (speedup 1.0000x reference)
import functools

import jax
import jax.numpy as jnp
from jax import lax
from jax.experimental import pallas as pl
from jax.experimental.pallas import tpu as pltpu
from jax.experimental.pallas import tpu_sc as plsc

_H = 64
_B = 4096
_S = 200
_N = _B * _S
_NC = 2
_NS = 16
_NW = _NC * _NS
_PER_W = _N // _NW
_TW = 100
_GROUP = 400
_TPG = _GROUP // _TW
_GROUPS = _PER_W // _GROUP
_IDX_ROWS = _N // _TW
_IDX_PER_W = _PER_W // _TW
_LANES = 16


@functools.partial(
    pl.kernel,
    out_type=jax.ShapeDtypeStruct((_N, _H), jnp.float32),
    mesh=plsc.VectorSubcoreMesh(core_axis_name="c", subcore_axis_name="s"),
    compiler_params=pltpu.CompilerParams(use_tc_tiling_on_sc=False),
    scratch_types=[
        pltpu.VMEM((_IDX_PER_W, _TW), jnp.int32),
        pltpu.VMEM((_GROUP, _H), jnp.float32),
        pltpu.VMEM((_GROUP, _H), jnp.float32),
        pltpu.SemaphoreType.DMA,
    ],
)
def _emb_kernel(ids_hbm, word_hbm, pos_hbm, out_hbm, idx_v, buf, posrep, sem):
    wid = lax.axis_index("s") * _NC + lax.axis_index("c")
    row_base = wid * _PER_W

    pltpu.sync_copy(ids_hbm.at[pl.ds(wid * _IDX_PER_W, _IDX_PER_W)], idx_v)
    pltpu.sync_copy(pos_hbm, posrep.at[pl.ds(0, _S)])
    pltpu.sync_copy(pos_hbm, posrep.at[pl.ds(_S, _S)])

    def group_body(g, carry):
        descs = []
        for t in range(_TPG):
            descs.append(
                pltpu.async_copy(
                    word_hbm.at[idx_v.at[g * _TPG + t]],
                    buf.at[pl.ds(t * _TW, _TW)],
                    sem,
                )
            )
        for d in descs:
            d.wait()

        def row_body(r, rcarry):
            for c in range(_H // _LANES):
                sl = pl.ds(c * _LANES, _LANES)
                buf[r, sl] = buf[r, sl] + posrep[r, sl]
            return rcarry

        lax.fori_loop(0, _GROUP, row_body, 0)
        pltpu.sync_copy(
            buf, out_hbm.at[pl.ds(row_base + g * _GROUP, _GROUP)]
        )
        return carry

    lax.fori_loop(0, _GROUPS, group_body, 0)


def kernel(input_ids, word_table, pos_table):
    ids = input_ids.reshape(_IDX_ROWS, _TW).astype(jnp.int32)
    out = _emb_kernel(ids, word_table, pos_table)
    return out.reshape(_B, _S, _H)


# speedup vs baseline: 1.0891x; 1.0891x over previous
import functools

import jax
import jax.numpy as jnp
from jax import lax
from jax.experimental import pallas as pl
from jax.experimental.pallas import tpu as pltpu
from jax.experimental.pallas import tpu_sc as plsc

_H = 64
_B = 4096
_S = 200
_N = _B * _S
_NC = 2
_NS = 16
_NW = _NC * _NS
_PER_W = _N // _NW
_TW = 100
_GROUP = 200
_TPG = _GROUP // _TW
_GROUPS = _PER_W // _GROUP
_NBUF = 4
_ITERS = _GROUPS // _NBUF
_IDX_ROWS = _N // _TW
_IDX_PER_W = _PER_W // _TW
_LANES = 16


@functools.partial(
    pl.kernel,
    out_type=jax.ShapeDtypeStruct((_N, _H), jnp.float32),
    mesh=plsc.VectorSubcoreMesh(core_axis_name="c", subcore_axis_name="s"),
    compiler_params=pltpu.CompilerParams(use_tc_tiling_on_sc=False),
    scratch_types=[
        pltpu.VMEM((_IDX_PER_W, _TW), jnp.int32),
        pltpu.VMEM((_GROUP, _H), jnp.float32),
        pltpu.VMEM((_GROUP, _H), jnp.float32),
        pltpu.VMEM((_GROUP, _H), jnp.float32),
        pltpu.VMEM((_GROUP, _H), jnp.float32),
        pltpu.VMEM((_GROUP, _H), jnp.float32),
        pltpu.SemaphoreType.DMA((_NBUF,)),
        pltpu.SemaphoreType.DMA((_NBUF,)),
    ],
)
def _emb_kernel(ids_hbm, word_hbm, pos_hbm, out_hbm, idx_v,
                b0, b1, b2, b3, posv, sem_g, sem_w):
    bufs = [b0, b1, b2, b3]
    wid = lax.axis_index("s") * _NC + lax.axis_index("c")
    row_base = wid * _PER_W

    pltpu.sync_copy(ids_hbm.at[pl.ds(wid * _IDX_PER_W, _IDX_PER_W)], idx_v)
    pltpu.sync_copy(pos_hbm, posv)

    def fire_gathers(gg, p):
        for t in range(_TPG):
            pltpu.async_copy(
                word_hbm.at[idx_v.at[gg * _TPG + t]],
                bufs[p].at[pl.ds(t * _TW, _TW)],
                sem_g.at[p],
            )

    def wait_gathers(p):
        pltpu.make_async_copy(
            word_hbm.at[pl.ds(0, _GROUP)], bufs[p], sem_g.at[p]
        ).wait()

    def wait_writeback(p):
        pltpu.make_async_copy(
            bufs[p], out_hbm.at[pl.ds(0, _GROUP)], sem_w.at[p]
        ).wait()

    for p in range(_NBUF - 1):
        fire_gathers(p, p)

    def iter_body(i, carry):
        for p in range(_NBUF):
            gg = i * _NBUF + p
            wait_gathers(p)
            nxt = (p + _NBUF - 1) % _NBUF
            if p == 0:
                @pl.when(i >= 1)
                def _():
                    wait_writeback(nxt)
                fire_gathers(gg + _NBUF - 1, nxt)
            else:
                @pl.when(i < _ITERS - 1)
                def _():
                    wait_writeback(nxt)
                    fire_gathers(gg + _NBUF - 1, nxt)

            buf = bufs[p]

            @plsc.parallel_loop(0, _GROUP, unroll=2)
            def _(r):
                for c in range(_H // _LANES):
                    sl = pl.ds(c * _LANES, _LANES)
                    buf[r, sl] = buf[r, sl] + posv[r, sl]

            pltpu.async_copy(
                buf, out_hbm.at[pl.ds(row_base + gg * _GROUP, _GROUP)],
                sem_w.at[p],
            )
        return carry

    lax.fori_loop(0, _ITERS, iter_body, 0)

    for p in range(_NBUF):
        wait_writeback(p)


def kernel(input_ids, word_table, pos_table):
    ids = input_ids.reshape(_IDX_ROWS, _TW).astype(jnp.int32)
    out = _emb_kernel(ids, word_table, pos_table)
    return out.reshape(_B, _S, _H)


# speedup vs baseline: 1.5462x; 1.4197x over previous
import functools

import jax
import jax.numpy as jnp
from jax import lax
from jax.experimental import pallas as pl
from jax.experimental.pallas import tpu as pltpu
from jax.experimental.pallas import tpu_sc as plsc

_H = 64
_B = 4096
_S = 200
_N = _B * _S
_NC = 2
_NS = 16
_NW = _NC * _NS
_PER_W = _N // _NW
_GROUP = 128
_GROUPS = _PER_W // _GROUP
_IDX_ROWS = _N // _GROUP
_NBUF = 4
_ITERS = _GROUPS // _NBUF
_LANES = 16


@functools.partial(
    pl.kernel,
    out_type=jax.ShapeDtypeStruct((_N, _H), jnp.float32),
    mesh=plsc.VectorSubcoreMesh(core_axis_name="c", subcore_axis_name="s"),
    compiler_params=pltpu.CompilerParams(use_tc_tiling_on_sc=True),
    scratch_types=[
        pltpu.VMEM((_GROUPS, _GROUP), jnp.int32),
        pltpu.VMEM((_GROUP, _H), jnp.float32),
        pltpu.VMEM((_GROUP, _H), jnp.float32),
        pltpu.VMEM((_GROUP, _H), jnp.float32),
        pltpu.VMEM((_GROUP, _H), jnp.float32),
        pltpu.VMEM((_S, _H), jnp.float32),
        pltpu.SemaphoreType.DMA((_NBUF,)),
        pltpu.SemaphoreType.DMA((_NBUF,)),
    ],
)
def _emb_kernel(ids_hbm, word_hbm, pos_hbm, out_hbm, idx_v,
                b0, b1, b2, b3, posv, sem_g, sem_w):
    bufs = [b0, b1, b2, b3]
    wid = lax.axis_index("s") * _NC + lax.axis_index("c")
    row_base = wid * _PER_W

    pltpu.sync_copy(ids_hbm.at[pl.ds(wid * _GROUPS, _GROUPS)], idx_v)
    pltpu.sync_copy(pos_hbm, posv)

    def fire_gathers(gg, p):
        buf = bufs[p]
        for k in range(_GROUP // _LANES):
            v = idx_v[gg, pl.ds(k * _LANES, _LANES)]
            for j in range(_LANES):
                pltpu.async_copy(
                    word_hbm.at[v[j]], buf.at[k * _LANES + j], sem_g.at[p]
                )

    def wait_gathers(p):
        pltpu.make_async_copy(
            out_hbm.at[pl.ds(0, _GROUP)], bufs[p], sem_g.at[p]
        ).wait()

    def wait_writeback(p):
        pltpu.make_async_copy(
            bufs[p], out_hbm.at[pl.ds(0, _GROUP)], sem_w.at[p]
        ).wait()

    def add_pos(gg, p):
        buf = bufs[p]
        pbase = lax.rem(gg * _GROUP, _S)

        @plsc.parallel_loop(0, _GROUP)
        def _(r):
            s = pbase + r
            s = jnp.where(s >= _S, s - _S, s)
            for c in range(_H // _LANES):
                sl = pl.ds(c * _LANES, _LANES)
                buf[r, sl] = buf[r, sl] + posv[s, sl]

    for p in range(_NBUF - 1):
        fire_gathers(p, p)

    def iter_body(i, carry):
        for p in range(_NBUF):
            gg = i * _NBUF + p
            nxt = (p + _NBUF - 1) % _NBUF
            if p == 0:
                @pl.when(i >= 1)
                def _():
                    wait_writeback(nxt)
                fire_gathers(gg + _NBUF - 1, nxt)
            else:
                @pl.when(i < _ITERS - 1)
                def _():
                    wait_writeback(nxt)
                    fire_gathers(gg + _NBUF - 1, nxt)

            wait_gathers(p)
            add_pos(gg, p)
            pltpu.async_copy(
                bufs[p], out_hbm.at[pl.ds(row_base + gg * _GROUP, _GROUP)],
                sem_w.at[p],
            )
        return carry

    lax.fori_loop(0, _ITERS, iter_body, 0)

    for p in range(_NBUF):
        wait_writeback(p)


def kernel(input_ids, word_table, pos_table):
    ids = input_ids.reshape(_IDX_ROWS, _GROUP).astype(jnp.int32)
    out = _emb_kernel(ids, word_table, pos_table)
    return out.reshape(_B, _S, _H)
